# Initial kernel scaffold; baseline (speedup 1.0000x reference)
import jax
import jax.numpy as jnp
from jax.experimental import pallas as pl


def kernel(pred_word_fg, pred_word_tblr, pred_word_orient, pred_char_fg, pred_char_tblr, pred_char_cls, im_scale_w, im_scale_h, original_im_w, original_im_h):
    raise NotImplementedError("write your pallas kernel here")


# speedup vs baseline: 26.4474x; 26.4474x over previous
import jax
import jax.numpy as jnp
from jax import lax
from jax.experimental import pallas as pl
from jax.experimental.pallas import tpu as pltpu

_WORD_MIN = 0.4
_WORD_IOU = 0.5
_CHAR_MIN = 0.35
_CHAR_IOU = 0.3
_STRIDE = 4.0
_KW = 1024
_KC = 2048
_H = 128
_W = 128
_HW = _H * _W
_NCLS = 68
_CH = 1024
_B = 256


def _mask_kernel(fgw_ref, fgc_ref, mw_ref, mc_ref):
    fgw = fgw_ref[...]
    fgc = fgc_ref[...]
    mw_ref[...] = jnp.where(fgw > _WORD_MIN, fgw, -1.0)
    mc_ref[...] = jnp.where((fgc > _CHAR_MIN) & (fgw > _WORD_MIN), fgc, -1.0)


def _masked_scores(fgw, fgc):
    return pl.pallas_call(
        _mask_kernel,
        out_shape=[
            jax.ShapeDtypeStruct((_H, _W), jnp.float32),
            jax.ShapeDtypeStruct((_H, _W), jnp.float32),
        ],
    )(fgw, fgc)


def _nms_kernel(mapsT_ref, idx_ref, sc_ref, par_ref, out_ref, cls_ref,
                over_ref, *, k, nrows, min_score, thresh, has_cls):
    f32 = jnp.float32
    idx = idx_ref[...]
    scores = sc_ref[...]

    acc = jnp.zeros((nrows, k), f32)
    for c in range(_HW // _CH):
        base = c * _CH
        rows_i = lax.broadcasted_iota(jnp.int32, (_CH, k), 0) + base
        onehot = (rows_i == idx).astype(f32)
        m = mapsT_ref[:, base:base + _CH]
        acc = acc + lax.dot_general(m, onehot, (((1,), (0,)), ((), ())),
                                    precision=lax.Precision.HIGHEST,
                                    preferred_element_type=f32)

    t = acc[0:1, :]
    b = acc[1:2, :]
    l = acc[2:3, :]
    r_ = acc[3:4, :]
    if has_cls:
        co = jnp.ones((1, k), f32)
        si = jnp.zeros((1, k), f32)
    else:
        co = acc[4:5, :]
        si = acc[5:6, :]

    sw = par_ref[0:1, 0:1]
    sh = par_ref[0:1, 1:2]
    imw = par_ref[0:1, 2:3]
    imh = par_ref[0:1, 3:4]

    xs = (idx & (_W - 1)).astype(f32)
    ys = (idx >> 7).astype(f32)

    x1 = sw * _STRIDE * (xs - l)
    y1 = sh * _STRIDE * (ys - t)
    x2 = sw * _STRIDE * (xs + r_)
    y2 = sh * _STRIDE * (ys + b)
    ax = sw * _STRIDE * xs
    ay = sh * _STRIDE * ys

    def rot(px, py):
        return ((px - ax) * co - (py - ay) * si + ax,
                (px - ax) * si + (py - ay) * co + ay)

    p1x, p1y = rot(x1, y1)
    p2x, p2y = rot(x2, y1)
    p3x, p3y = rot(x2, y2)
    p4x, p4y = rot(x1, y2)

    xmn = jnp.minimum(jnp.minimum(p1x, p2x), jnp.minimum(p3x, p4x))
    xmx = jnp.maximum(jnp.maximum(p1x, p2x), jnp.maximum(p3x, p4x))
    ymn = jnp.minimum(jnp.minimum(p1y, p2y), jnp.minimum(p3y, p4y))
    ymx = jnp.maximum(jnp.maximum(p1y, p2y), jnp.maximum(p3y, p4y))
    area = jnp.clip(xmx - xmn, 0.0) * jnp.clip(ymx - ymn, 0.0)

    ident = (lax.broadcasted_iota(jnp.int32, (_B, _B), 0) ==
             lax.broadcasted_iota(jnp.int32, (_B, _B), 1)).astype(f32)

    def col(row, rb):
        blk = row[:, rb * _B:(rb + 1) * _B]
        return lax.dot_general(ident, blk, (((1,), (1,)), ((), ())),
                               precision=lax.Precision.HIGHEST,
                               preferred_element_type=f32)

    for rb in range(k // _B):
        x1c = col(xmn, rb)
        y1c = col(ymn, rb)
        x2c = col(xmx, rb)
        y2c = col(ymx, rb)
        ac = col(area, rb)
        ix1 = jnp.maximum(x1c, xmn)
        iy1 = jnp.maximum(y1c, ymn)
        ix2 = jnp.minimum(x2c, xmx)
        iy2 = jnp.minimum(y2c, ymx)
        inter = jnp.clip(ix2 - ix1, 0.0) * jnp.clip(iy2 - iy1, 0.0)
        denom = ac + area - inter + 1e-9
        over_ref[rb * _B:(rb + 1) * _B, :] = (inter > thresh * denom).astype(f32)

    validf = (scores > min_score)
    lane = lax.broadcasted_iota(jnp.int32, (1, k), 1)

    def body(rr, keep):
        row = over_ref[pl.ds(rr, 1), :]
        sup = jnp.sum(row * keep)
        return jnp.where((lane == rr) & validf & (sup <= 0.0), 1.0, keep)

    keep = lax.fori_loop(0, k, body, jnp.zeros((1, k), f32))

    def cc(p, bound):
        return jnp.clip(jnp.round(p), 0.0, bound - 1.0)

    rows_out = jnp.concatenate(
        [cc(p1x, imw), cc(p1y, imh), cc(p2x, imw), cc(p2y, imh),
         cc(p3x, imw), cc(p3y, imh), cc(p4x, imw), cc(p4y, imh),
         scores, jnp.zeros((7, k), f32)], axis=0)
    out_ref[...] = rows_out * keep

    if has_cls:
        cls_rows = acc[4:4 + _NCLS, :] * keep
        cls_ref[...] = jnp.concatenate(
            [cls_rows, jnp.zeros((4, k), f32)], axis=0)


def _nms_call(mapsT, idx, scores, par, *, k, nrows, min_score, thresh,
              has_cls):
    import functools
    body = functools.partial(_nms_kernel, k=k, nrows=nrows,
                             min_score=min_score, thresh=thresh,
                             has_cls=has_cls)
    out_shapes = [jax.ShapeDtypeStruct((16, k), jnp.float32)]
    if has_cls:
        out_shapes.append(jax.ShapeDtypeStruct((_NCLS + 4, k), jnp.float32))
    else:
        out_shapes.append(jax.ShapeDtypeStruct((8, k), jnp.float32))
    return pl.pallas_call(
        body,
        out_shape=out_shapes,
        scratch_shapes=[pltpu.VMEM((k, k), jnp.float32)],
    )(mapsT, idx, scores, par)


def kernel(pred_word_fg, pred_word_tblr, pred_word_orient, pred_char_fg,
           pred_char_tblr, pred_char_cls, im_scale_w, im_scale_h,
           original_im_w, original_im_h):
    f32 = jnp.float32
    sw = jnp.float32(im_scale_w)
    sh = jnp.float32(im_scale_h)
    imw = jnp.float32(original_im_w)
    imh = jnp.float32(original_im_h)
    par = jnp.concatenate([jnp.stack([sw, sh, imw, imh]),
                           jnp.zeros((124,), f32)]).reshape(1, 128)

    mw, mc = _masked_scores(pred_word_fg, pred_char_fg)
    ws, wi = lax.top_k(mw.reshape(-1), _KW)
    cs, ci = lax.top_k(mc.reshape(-1), _KC)

    co = jnp.cos(pred_word_orient).reshape(1, -1)
    si = jnp.sin(pred_word_orient).reshape(1, -1)
    mapsT_w = jnp.concatenate(
        [pred_word_tblr.reshape(4, -1), co, si, jnp.zeros((2, _HW), f32)], 0)
    mapsT_c = jnp.concatenate(
        [pred_char_tblr.reshape(4, -1), pred_char_cls.reshape(_NCLS, -1)], 0)

    w16, _ = _nms_call(mapsT_w, wi.reshape(1, -1), ws.reshape(1, -1), par,
                       k=_KW, nrows=8, min_score=_WORD_MIN,
                       thresh=_WORD_IOU, has_cls=False)
    c16, cls72 = _nms_call(mapsT_c, ci.reshape(1, -1), cs.reshape(1, -1),
                           par, k=_KC, nrows=4 + _NCLS,
                           min_score=_CHAR_MIN, thresh=_CHAR_IOU,
                           has_cls=True)

    word_boxes = w16[:9].T
    char_boxes = c16[:9].T
    char_scores = cls72[:_NCLS].T
    return (char_boxes, char_scores, word_boxes)


# speedup vs baseline: 29.2539x; 1.1061x over previous
import functools

import jax
import jax.numpy as jnp
from jax import lax
from jax.experimental import pallas as pl
from jax.experimental.pallas import tpu as pltpu
from jax.experimental.pallas import tpu_sc as plsc

_WORD_MIN = 0.4
_WORD_IOU = 0.5
_CHAR_MIN = 0.35
_CHAR_IOU = 0.3
_STRIDE = 4.0
_KW = 1024
_KC = 2048
_H = 128
_W = 128
_HW = _H * _W
_NCLS = 68
_CH = 1024
_B = 256


def _mask_kernel(fgw_ref, fgc_ref, mw_ref, mc_ref):
    fgw = fgw_ref[...]
    fgc = fgc_ref[...]
    mw_ref[...] = jnp.where(fgw > _WORD_MIN, fgw, -1.0)
    mc_ref[...] = jnp.where((fgc > _CHAR_MIN) & (fgw > _WORD_MIN), fgc, -1.0)


def _masked_scores(fgw, fgc):
    return pl.pallas_call(
        _mask_kernel,
        out_shape=[
            jax.ShapeDtypeStruct((_H, _W), jnp.float32),
            jax.ShapeDtypeStruct((_H, _W), jnp.float32),
        ],
    )(fgw, fgc)


_SC_NC = 2
_SC_NS = 16


def _sc_gather(table, idx, d):
    b = idx.shape[0]
    nw = _SC_NC * _SC_NS
    b_per_w = b // nw
    mesh = plsc.VectorSubcoreMesh(core_axis_name="c", subcore_axis_name="s")

    @functools.partial(
        pl.kernel, mesh=mesh,
        out_type=jax.ShapeDtypeStruct((b, d), jnp.float32),
        scratch_types=[
            pltpu.VMEM((b_per_w,), jnp.int32),
            pltpu.VMEM((b_per_w, d), jnp.float32),
            pltpu.SemaphoreType.DMA,
        ],
    )
    def gat(table_hbm, idx_hbm, out_hbm, idx_v, rows_v, sem):
        wid = lax.axis_index("s") * _SC_NC + lax.axis_index("c")
        base = wid * b_per_w
        pltpu.sync_copy(idx_hbm.at[pl.ds(base, b_per_w)], idx_v)
        pltpu.async_copy(table_hbm.at[idx_v], rows_v, sem).wait()
        pltpu.sync_copy(rows_v, out_hbm.at[pl.ds(base, b_per_w)])

    return gat(table, idx)


def _nms_kernel(gat_ref, idx_ref, sc_ref, par_ref, out_ref, cls_ref,
                over_ref, *, k, nrows, min_score, thresh, has_cls):
    f32 = jnp.float32
    idx = idx_ref[...]
    scores = sc_ref[...]

    acc = gat_ref[...]

    t = acc[0:1, :]
    b = acc[1:2, :]
    l = acc[2:3, :]
    r_ = acc[3:4, :]
    if has_cls:
        co = jnp.ones((1, k), f32)
        si = jnp.zeros((1, k), f32)
    else:
        co = acc[4:5, :]
        si = acc[5:6, :]

    sw = par_ref[0:1, 0:1]
    sh = par_ref[0:1, 1:2]
    imw = par_ref[0:1, 2:3]
    imh = par_ref[0:1, 3:4]

    xs = (idx & (_W - 1)).astype(f32)
    ys = (idx >> 7).astype(f32)

    x1 = sw * _STRIDE * (xs - l)
    y1 = sh * _STRIDE * (ys - t)
    x2 = sw * _STRIDE * (xs + r_)
    y2 = sh * _STRIDE * (ys + b)
    ax = sw * _STRIDE * xs
    ay = sh * _STRIDE * ys

    def rot(px, py):
        return ((px - ax) * co - (py - ay) * si + ax,
                (px - ax) * si + (py - ay) * co + ay)

    p1x, p1y = rot(x1, y1)
    p2x, p2y = rot(x2, y1)
    p3x, p3y = rot(x2, y2)
    p4x, p4y = rot(x1, y2)

    xmn = jnp.minimum(jnp.minimum(p1x, p2x), jnp.minimum(p3x, p4x))
    xmx = jnp.maximum(jnp.maximum(p1x, p2x), jnp.maximum(p3x, p4x))
    ymn = jnp.minimum(jnp.minimum(p1y, p2y), jnp.minimum(p3y, p4y))
    ymx = jnp.maximum(jnp.maximum(p1y, p2y), jnp.maximum(p3y, p4y))
    area = jnp.clip(xmx - xmn, 0.0) * jnp.clip(ymx - ymn, 0.0)

    ident = (lax.broadcasted_iota(jnp.int32, (_B, _B), 0) ==
             lax.broadcasted_iota(jnp.int32, (_B, _B), 1)).astype(f32)

    def col(row, rb):
        blk = row[:, rb * _B:(rb + 1) * _B]
        return lax.dot_general(ident, blk, (((1,), (1,)), ((), ())),
                               precision=lax.Precision.HIGHEST,
                               preferred_element_type=f32)

    for rb in range(k // _B):
        x1c = col(xmn, rb)
        y1c = col(ymn, rb)
        x2c = col(xmx, rb)
        y2c = col(ymx, rb)
        ac = col(area, rb)
        ix1 = jnp.maximum(x1c, xmn)
        iy1 = jnp.maximum(y1c, ymn)
        ix2 = jnp.minimum(x2c, xmx)
        iy2 = jnp.minimum(y2c, ymx)
        inter = jnp.clip(ix2 - ix1, 0.0) * jnp.clip(iy2 - iy1, 0.0)
        denom = ac + area - inter + 1e-9
        over_ref[rb * _B:(rb + 1) * _B, :] = (inter > thresh * denom).astype(f32)

    validf = (scores > min_score)
    lane = lax.broadcasted_iota(jnp.int32, (1, k), 1)

    def body(rr, keep):
        row = over_ref[pl.ds(rr, 1), :]
        sup = jnp.sum(row * keep)
        return jnp.where((lane == rr) & validf & (sup <= 0.0), 1.0, keep)

    keep = lax.fori_loop(0, k, body, jnp.zeros((1, k), f32))

    def cc(p, bound):
        return jnp.clip(jnp.round(p), 0.0, bound - 1.0)

    rows_out = jnp.concatenate(
        [cc(p1x, imw), cc(p1y, imh), cc(p2x, imw), cc(p2y, imh),
         cc(p3x, imw), cc(p3y, imh), cc(p4x, imw), cc(p4y, imh),
         scores, jnp.zeros((7, k), f32)], axis=0)
    out_ref[...] = rows_out * keep

    if has_cls:
        cls_rows = acc[4:4 + _NCLS, :] * keep
        cls_ref[...] = jnp.concatenate(
            [cls_rows, jnp.zeros((4, k), f32)], axis=0)


def _nms_call(gat, idx, scores, par, *, k, nrows, min_score, thresh,
              has_cls):
    body = functools.partial(_nms_kernel, k=k, nrows=nrows,
                             min_score=min_score, thresh=thresh,
                             has_cls=has_cls)
    out_shapes = [jax.ShapeDtypeStruct((16, k), jnp.float32)]
    if has_cls:
        out_shapes.append(jax.ShapeDtypeStruct((_NCLS + 4, k), jnp.float32))
    else:
        out_shapes.append(jax.ShapeDtypeStruct((8, k), jnp.float32))
    return pl.pallas_call(
        body,
        out_shape=out_shapes,
        scratch_shapes=[pltpu.VMEM((k, k), jnp.float32)],
    )(gat, idx, scores, par)


def kernel(pred_word_fg, pred_word_tblr, pred_word_orient, pred_char_fg,
           pred_char_tblr, pred_char_cls, im_scale_w, im_scale_h,
           original_im_w, original_im_h):
    f32 = jnp.float32
    sw = jnp.float32(im_scale_w)
    sh = jnp.float32(im_scale_h)
    imw = jnp.float32(original_im_w)
    imh = jnp.float32(original_im_h)
    par = jnp.concatenate([jnp.stack([sw, sh, imw, imh]),
                           jnp.zeros((124,), f32)]).reshape(1, 128)

    mw, mc = _masked_scores(pred_word_fg, pred_char_fg)
    ws, wi = lax.top_k(mw.reshape(-1), _KW)
    cs, ci = lax.top_k(mc.reshape(-1), _KC)

    co = jnp.cos(pred_word_orient).reshape(1, -1)
    si = jnp.sin(pred_word_orient).reshape(1, -1)
    table_w = jnp.concatenate(
        [pred_word_tblr.reshape(4, -1), co, si,
         jnp.zeros((122, _HW), f32)], 0).T
    table_c = jnp.concatenate(
        [pred_char_tblr.reshape(4, -1), pred_char_cls.reshape(_NCLS, -1),
         jnp.zeros((56, _HW), f32)], 0).T

    gat_w = _sc_gather(table_w, wi, 128).T
    gat_c = _sc_gather(table_c, ci, 128).T

    w16, _ = _nms_call(gat_w, wi.reshape(1, -1), ws.reshape(1, -1), par,
                       k=_KW, nrows=128, min_score=_WORD_MIN,
                       thresh=_WORD_IOU, has_cls=False)
    c16, cls72 = _nms_call(gat_c, ci.reshape(1, -1), cs.reshape(1, -1),
                           par, k=_KC, nrows=128,
                           min_score=_CHAR_MIN, thresh=_CHAR_IOU,
                           has_cls=True)

    word_boxes = w16[:9].T
    char_boxes = c16[:9].T
    char_scores = cls72[:_NCLS].T
    return (char_boxes, char_scores, word_boxes)


# speedup vs baseline: 32.3586x; 1.1061x over previous
import functools

import jax
import jax.numpy as jnp
from jax import lax
from jax.experimental import pallas as pl
from jax.experimental.pallas import tpu as pltpu
from jax.experimental.pallas import tpu_sc as plsc

_WORD_MIN = 0.4
_WORD_IOU = 0.5
_CHAR_MIN = 0.35
_CHAR_IOU = 0.3
_STRIDE = 4.0
_KW = 1024
_KC = 2048
_H = 128
_W = 128
_HW = _H * _W
_NCLS = 68
_CH = 1024
_B = 256


def _mask_kernel(fgw_ref, fgc_ref, mw_ref, mc_ref):
    fgw = fgw_ref[...]
    fgc = fgc_ref[...]
    mw_ref[...] = jnp.where(fgw > _WORD_MIN, fgw, -1.0)
    mc_ref[...] = jnp.where((fgc > _CHAR_MIN) & (fgw > _WORD_MIN), fgc, -1.0)


def _masked_scores(fgw, fgc):
    return pl.pallas_call(
        _mask_kernel,
        out_shape=[
            jax.ShapeDtypeStruct((_H, _W), jnp.float32),
            jax.ShapeDtypeStruct((_H, _W), jnp.float32),
        ],
    )(fgw, fgc)


_SC_NC = 2
_SC_NS = 16


def _sc_gather(table, idx, d):
    b = idx.shape[0]
    nw = _SC_NC * _SC_NS
    b_per_w = b // nw
    mesh = plsc.VectorSubcoreMesh(core_axis_name="c", subcore_axis_name="s")

    @functools.partial(
        pl.kernel, mesh=mesh,
        out_type=jax.ShapeDtypeStruct((b, d), jnp.float32),
        scratch_types=[
            pltpu.VMEM((b_per_w,), jnp.int32),
            pltpu.VMEM((b_per_w, d), jnp.float32),
            pltpu.SemaphoreType.DMA,
        ],
    )
    def gat(table_hbm, idx_hbm, out_hbm, idx_v, rows_v, sem):
        wid = lax.axis_index("s") * _SC_NC + lax.axis_index("c")
        base = wid * b_per_w
        pltpu.sync_copy(idx_hbm.at[pl.ds(base, b_per_w)], idx_v)
        pltpu.async_copy(table_hbm.at[idx_v], rows_v, sem).wait()
        pltpu.sync_copy(rows_v, out_hbm.at[pl.ds(base, b_per_w)])

    return gat(table, idx)


def _nms_kernel(gat_ref, idx_ref, sc_ref, par_ref, out_ref, cls_ref,
                over_ref, *, k, nrows, min_score, thresh, has_cls):
    f32 = jnp.float32
    idx = idx_ref[...]
    scores = sc_ref[...]

    acc = gat_ref[...]

    t = acc[0:1, :]
    b = acc[1:2, :]
    l = acc[2:3, :]
    r_ = acc[3:4, :]
    if has_cls:
        co = jnp.ones((1, k), f32)
        si = jnp.zeros((1, k), f32)
    else:
        co = acc[4:5, :]
        si = acc[5:6, :]

    sw = par_ref[0:1, 0:1]
    sh = par_ref[0:1, 1:2]
    imw = par_ref[0:1, 2:3]
    imh = par_ref[0:1, 3:4]

    xs = (idx & (_W - 1)).astype(f32)
    ys = (idx >> 7).astype(f32)

    x1 = sw * _STRIDE * (xs - l)
    y1 = sh * _STRIDE * (ys - t)
    x2 = sw * _STRIDE * (xs + r_)
    y2 = sh * _STRIDE * (ys + b)
    ax = sw * _STRIDE * xs
    ay = sh * _STRIDE * ys

    def rot(px, py):
        return ((px - ax) * co - (py - ay) * si + ax,
                (px - ax) * si + (py - ay) * co + ay)

    p1x, p1y = rot(x1, y1)
    p2x, p2y = rot(x2, y1)
    p3x, p3y = rot(x2, y2)
    p4x, p4y = rot(x1, y2)

    xmn = jnp.minimum(jnp.minimum(p1x, p2x), jnp.minimum(p3x, p4x))
    xmx = jnp.maximum(jnp.maximum(p1x, p2x), jnp.maximum(p3x, p4x))
    ymn = jnp.minimum(jnp.minimum(p1y, p2y), jnp.minimum(p3y, p4y))
    ymx = jnp.maximum(jnp.maximum(p1y, p2y), jnp.maximum(p3y, p4y))
    area = jnp.clip(xmx - xmn, 0.0) * jnp.clip(ymx - ymn, 0.0)

    ident = (lax.broadcasted_iota(jnp.int32, (_B, _B), 0) ==
             lax.broadcasted_iota(jnp.int32, (_B, _B), 1)).astype(f32)

    def col(row, rb):
        blk = row[:, rb * _B:(rb + 1) * _B]
        return lax.dot_general(ident, blk, (((1,), (1,)), ((), ())),
                               precision=lax.Precision.HIGHEST,
                               preferred_element_type=f32)

    for rb in range(k // _B):
        x1c = col(xmn, rb)
        y1c = col(ymn, rb)
        x2c = col(xmx, rb)
        y2c = col(ymx, rb)
        ac = col(area, rb)
        ix1 = jnp.maximum(x1c, xmn)
        iy1 = jnp.maximum(y1c, ymn)
        ix2 = jnp.minimum(x2c, xmx)
        iy2 = jnp.minimum(y2c, ymx)
        inter = jnp.clip(ix2 - ix1, 0.0) * jnp.clip(iy2 - iy1, 0.0)
        denom = ac + area - inter + 1e-9
        over_ref[rb * _B:(rb + 1) * _B, :] = (inter > thresh * denom).astype(f32)

    validf = (scores > min_score)
    lane_b = lax.broadcasted_iota(jnp.int32, (1, _B), 1)
    done = []
    for bi in range(k // _B):
        base = bi * _B
        validb = validf[:, base:base + _B]
        if bi > 0:
            keep_prior = jnp.concatenate(done, axis=1)
            over_prior = over_ref[base:base + _B, 0:base]
            ext = lax.dot_general(over_prior, keep_prior,
                                  (((1,), (1,)), ((), ())),
                                  preferred_element_type=f32)
            ext_row = lax.dot_general(ext, ident,
                                      (((0,), (0,)), ((), ())),
                                      preferred_element_type=f32)
            validb = validb & (ext_row <= 0.0)

        def body(rr, keepb, base=base, validb=validb):
            row = over_ref[pl.ds(base + rr, 1), base:base + _B]
            sup = jnp.sum(row * keepb)
            return jnp.where((lane_b == rr) & validb & (sup <= 0.0),
                             1.0, keepb)

        done.append(lax.fori_loop(0, _B, body, jnp.zeros((1, _B), f32)))
    keep = jnp.concatenate(done, axis=1)

    def cc(p, bound):
        return jnp.clip(jnp.round(p), 0.0, bound - 1.0)

    rows_out = jnp.concatenate(
        [cc(p1x, imw), cc(p1y, imh), cc(p2x, imw), cc(p2y, imh),
         cc(p3x, imw), cc(p3y, imh), cc(p4x, imw), cc(p4y, imh),
         scores, jnp.zeros((7, k), f32)], axis=0)
    out_ref[...] = rows_out * keep

    if has_cls:
        cls_rows = acc[4:4 + _NCLS, :] * keep
        cls_ref[...] = jnp.concatenate(
            [cls_rows, jnp.zeros((4, k), f32)], axis=0)


def _nms_call(gat, idx, scores, par, *, k, nrows, min_score, thresh,
              has_cls):
    body = functools.partial(_nms_kernel, k=k, nrows=nrows,
                             min_score=min_score, thresh=thresh,
                             has_cls=has_cls)
    out_shapes = [jax.ShapeDtypeStruct((16, k), jnp.float32)]
    if has_cls:
        out_shapes.append(jax.ShapeDtypeStruct((_NCLS + 4, k), jnp.float32))
    else:
        out_shapes.append(jax.ShapeDtypeStruct((8, k), jnp.float32))
    return pl.pallas_call(
        body,
        out_shape=out_shapes,
        scratch_shapes=[pltpu.VMEM((k, k), jnp.float32)],
    )(gat, idx, scores, par)


def kernel(pred_word_fg, pred_word_tblr, pred_word_orient, pred_char_fg,
           pred_char_tblr, pred_char_cls, im_scale_w, im_scale_h,
           original_im_w, original_im_h):
    f32 = jnp.float32
    sw = jnp.float32(im_scale_w)
    sh = jnp.float32(im_scale_h)
    imw = jnp.float32(original_im_w)
    imh = jnp.float32(original_im_h)
    par = jnp.concatenate([jnp.stack([sw, sh, imw, imh]),
                           jnp.zeros((124,), f32)]).reshape(1, 128)

    mw, mc = _masked_scores(pred_word_fg, pred_char_fg)
    ws, wi = lax.top_k(mw.reshape(-1), _KW)
    cs, ci = lax.top_k(mc.reshape(-1), _KC)

    co = jnp.cos(pred_word_orient).reshape(1, -1)
    si = jnp.sin(pred_word_orient).reshape(1, -1)
    table_w = jnp.concatenate(
        [pred_word_tblr.reshape(4, -1), co, si,
         jnp.zeros((122, _HW), f32)], 0).T
    table_c = jnp.concatenate(
        [pred_char_tblr.reshape(4, -1), pred_char_cls.reshape(_NCLS, -1),
         jnp.zeros((56, _HW), f32)], 0).T

    gat_w = _sc_gather(table_w, wi, 128).T
    gat_c = _sc_gather(table_c, ci, 128).T

    w16, _ = _nms_call(gat_w, wi.reshape(1, -1), ws.reshape(1, -1), par,
                       k=_KW, nrows=128, min_score=_WORD_MIN,
                       thresh=_WORD_IOU, has_cls=False)
    c16, cls72 = _nms_call(gat_c, ci.reshape(1, -1), cs.reshape(1, -1),
                           par, k=_KC, nrows=128,
                           min_score=_CHAR_MIN, thresh=_CHAR_IOU,
                           has_cls=True)

    word_boxes = w16[:9].T
    char_boxes = c16[:9].T
    char_scores = cls72[:_NCLS].T
    return (char_boxes, char_scores, word_boxes)
